# baseline (speedup 1.0000x reference)
import jax
import jax.numpy as jnp
from jax import lax
from jax.experimental import pallas as pl
from jax.experimental.pallas import tpu as pltpu
from jax.experimental.pallas import tpu_sc as plsc

N = 10000
E = 320000
DIN = 128
DH = 256
DOUT = 128
NH = 10000

NC = 2
NS = 16
K = 80
CNT_PAD = 10240
ZBLK = K
NBLK = NH // ZBLK

_mesh = plsc.VectorSubcoreMesh(
    core_axis_name="c", subcore_axis_name="s", num_cores=NC, num_subcores=NS
)


RB = 4
IB = 8
LEAD = 2


def _make_sc_pass(width, split_features, count_dst=False):
    ept = (E // NS) if split_features else (E // (NC * NS))
    nchunks = ept // K
    nsteps = ((nchunks + IB - 1) // IB) * IB

    out_type = [jax.ShapeDtypeStruct((NC * NH, width), jnp.float32)]
    if count_dst:
        out_type.append(jax.ShapeDtypeStruct((NC * CNT_PAD,), jnp.float32))
    out_type = tuple(out_type) if count_dst else out_type[0]
    scratch = (
        [pltpu.VMEM_SHARED((NH, width), jnp.float32)]
        + ([pltpu.VMEM_SHARED((CNT_PAD,), jnp.float32)] if count_dst else [])
        + [pltpu.VMEM((K, width), jnp.float32) for _ in range(RB)]
        + [pltpu.VMEM((K,), jnp.int32) for _ in range(IB)]
        + [pltpu.VMEM((K,), jnp.int32) for _ in range(IB)]
        + ([pltpu.VMEM((K,), jnp.float32)] if count_dst else [])
        + [pltpu.SemaphoreType.DMA for _ in range(2 * RB + IB)]
    )

    def body(*args):
        if count_dst:
            (table, src_hbm, dst_hbm, ones_hbm, zero_hbm,
             out, cnt_out, acc, cnt, *rest) = args
        else:
            table, src_hbm, dst_hbm, out, acc, *rest = args
            cnt = cnt_out = ones_hbm = zero_hbm = None
        rows = rest[:RB]
        sidx = rest[RB:RB + IB]
        didx = rest[RB + IB:RB + 2 * IB]
        if count_dst:
            onesb = rest[RB + 2 * IB]
            rest = rest[:RB + 2 * IB] + rest[RB + 2 * IB + 1:]
        gsem = rest[RB + 2 * IB:RB + 2 * IB + RB]
        ssem = rest[RB + 2 * IB + RB:RB + 2 * IB + 2 * RB]
        isem = rest[RB + 2 * IB + 2 * RB:]
        c = lax.axis_index("c")
        s = lax.axis_index("s")
        z16 = jnp.zeros((16,), jnp.float32)
        zbuf = rows[0]

        def zrow(i, carry):
            for k in range(width // 16):
                zbuf[i, pl.ds(k * 16, 16)] = z16
            return carry

        lax.fori_loop(0, ZBLK, zrow, 0)

        zsem = rest[RB + 2 * IB]

        def zacc(k, carry):
            b = s + k * NS

            @pl.when(b < NBLK)
            def _():
                pltpu.async_copy(zbuf, acc.at[pl.ds(b * ZBLK, ZBLK)], zsem)

            return carry

        lax.fori_loop(0, (NBLK + NS - 1) // NS, zacc, 0)
        if count_dst:
            cb = CNT_PAD // NS
            pltpu.sync_copy(ones_hbm, onesb)
            pltpu.sync_copy(zero_hbm, cnt.at[pl.ds(s * cb, cb)])

        def zaccw(k, carry):
            b = s + k * NS

            @pl.when(b < NBLK)
            def _():
                pltpu.make_async_copy(
                    zbuf, acc.at[pl.ds(b * ZBLK, ZBLK)], zsem
                ).wait()

            return carry

        lax.fori_loop(0, (NBLK + NS - 1) // NS, zaccw, 0)
        plsc.subcore_barrier()

        if split_features:
            base_d = s * ept
            base_s = c * E + base_d
        else:
            base_d = (c * NS + s) * ept
            base_s = base_d

        def istart(t, i):
            pltpu.async_copy(
                src_hbm.at[pl.ds(base_s + t * K, K)], sidx[i], isem[i]
            )
            pltpu.async_copy(
                dst_hbm.at[pl.ds(base_d + t * K, K)], didx[i], isem[i]
            )

        def iwait(t, i):
            pltpu.make_async_copy(
                src_hbm.at[pl.ds(base_s + t * K, K)], sidx[i], isem[i]
            ).wait()
            pltpu.make_async_copy(
                dst_hbm.at[pl.ds(base_d + t * K, K)], didx[i], isem[i]
            ).wait()

        def gstart(i, b):
            pltpu.async_copy(table.at[sidx[i]], rows[b], gsem[b])

        def gwait(i, b):
            pltpu.make_async_copy(table.at[sidx[i]], rows[b], gsem[b]).wait()

        def sstart(i, b):
            pltpu.async_copy(rows[b], acc.at[didx[i]], ssem[b], add=True)
            if count_dst:
                pltpu.async_copy(onesb, cnt.at[didx[i]], ssem[b], add=True)

        def swait(i, b):
            pltpu.make_async_copy(
                rows[b], acc.at[didx[i]], ssem[b]
            ).wait()
            if count_dst:
                pltpu.make_async_copy(
                    onesb, cnt.at[didx[i]], ssem[b]
                ).wait()

        for t in range(IB // 2):
            istart(t, t)
        for t in range(IB // 2):
            iwait(t, t)
        for t in range(LEAD):
            gstart(t, t)

        def step(jo, carry):
            for b16 in range(IB):
                j = jo * IB + b16
                b8 = b16 % RB

                t2 = j + IB // 2
                i2 = (b16 + IB // 2) % IB

                @pl.when(t2 < nchunks)
                def _():
                    istart(t2, i2)

                @pl.when(j < nchunks)
                def _():
                    gwait(b16, b8)
                    sstart(b16, b8)

                tgt = j + LEAD
                bg = (b16 + LEAD) % RB
                ig = (b16 + LEAD) % IB

                @pl.when(tgt < nchunks)
                def _():
                    @pl.when(j >= LEAD)
                    def _():
                        swait((b16 + LEAD) % IB, bg)
                        iwait(tgt, ig)

                    gstart(ig, bg)

            return carry

        lax.fori_loop(0, nsteps // IB, step, 0)
        for t in range(nchunks - 2 * LEAD, nchunks):
            swait(t % IB, t % RB)
        plsc.subcore_barrier()
        if count_dst:
            pltpu.sync_copy(
                cnt.at[pl.ds(s * cb, cb)],
                cnt_out.at[pl.ds(c * CNT_PAD + s * cb, cb)],
            )

        def rd(k, carry):
            b = s + k * NS

            @pl.when(b < NBLK)
            def _():
                pltpu.async_copy(
                    acc.at[pl.ds(b * ZBLK, ZBLK)],
                    out.at[pl.ds(c * NH + b * ZBLK, ZBLK)],
                    zsem,
                )

            return carry

        lax.fori_loop(0, (NBLK + NS - 1) // NS, rd, 0)

        def rdw(k, carry):
            b = s + k * NS

            @pl.when(b < NBLK)
            def _():
                pltpu.make_async_copy(
                    acc.at[pl.ds(b * ZBLK, ZBLK)],
                    out.at[pl.ds(c * NH + b * ZBLK, ZBLK)],
                    zsem,
                ).wait()

            return carry

        lax.fori_loop(0, (NBLK + NS - 1) // NS, rdw, 0)

    return pl.kernel(
        body, out_type=out_type, mesh=_mesh, scratch_types=scratch
    )


_pass_s = _make_sc_pass(128, split_features=True, count_dst=True)
_pass_e = _make_sc_pass(128, split_features=False)


def _mm1_body(x_ref, w_ref, o_ref):
    xw = jnp.dot(x_ref[...], w_ref[...].T, preferred_element_type=jnp.float32)
    o_ref[0] = xw[:, :128]
    o_ref[1] = xw[:, 128:]


_mm1 = pl.pallas_call(
    _mm1_body,
    grid=(10,),
    in_specs=[
        pl.BlockSpec((N // 10, DIN), lambda i: (i, 0)),
        pl.BlockSpec((DH, DIN), lambda i: (0, 0)),
    ],
    out_specs=pl.BlockSpec((2, N // 10, 128), lambda i: (0, i, 0)),
    out_shape=jax.ShapeDtypeStruct((2, N, 128), jnp.float32),
)


def _inv(cnt_ref):
    c0 = cnt_ref[0, :NH]
    return jnp.where(c0 > 0, 1.0 / c0, 0.0)


def _scale_s_body(uf_ref, cnt_ref, o_ref):
    o_ref[...] = uf_ref[...] * _inv(cnt_ref)[None, :, None]


_scale_s = pl.pallas_call(
    _scale_s_body,
    out_shape=jax.ShapeDtypeStruct((2, NH, 128), jnp.float32),
)


def _scale_e_body(uf_ref, cnt_ref, o_ref):
    o_ref[...] = (uf_ref[0] + uf_ref[1]) * _inv(cnt_ref)[:, None]


_scale_e = pl.pallas_call(
    _scale_e_body,
    out_shape=jax.ShapeDtypeStruct((NH, 128), jnp.float32),
)


def _layer2_body(na_ref, cnt_ref, b_ref, w_ref, o_ref):
    h = jnp.concatenate([na_ref[0], na_ref[1]], axis=1)
    h = h * _inv(cnt_ref)[:, None] + b_ref[...]
    h = jnp.maximum(h, 0.0)
    o_ref[...] = jnp.dot(h, w_ref[...].T, preferred_element_type=jnp.float32)


_layer2 = pl.pallas_call(
    _layer2_body,
    out_shape=jax.ShapeDtypeStruct((N, DOUT), jnp.float32),
)


def _final_body(na_ref, cnt_ref, b_ref, o_ref):
    o_ref[...] = (na_ref[0] + na_ref[1]) * _inv(cnt_ref)[:, None] + b_ref[...]


_final = pl.pallas_call(
    _final_body,
    out_shape=jax.ShapeDtypeStruct((N, DOUT), jnp.float32),
)


def kernel(x, edge_index, W1, b1, W2, b2):
    node_idx = edge_index[0]
    hyper_idx = edge_index[1]

    ones_k = jnp.ones((K,), jnp.float32)
    zeros_cb = jnp.zeros((CNT_PAD // NS,), jnp.float32)

    node2 = jnp.concatenate([node_idx, node_idx + N])
    hyper2 = jnp.concatenate([hyper_idx, hyper_idx + NH])

    xw = _mm1(x, W1).reshape(2 * N, 128)
    uf, cntb = _pass_s(xw, node2, hyper_idx, ones_k, zeros_cb)
    cntb = cntb.reshape(NC, CNT_PAD)
    ef = _scale_s(uf.reshape(2, NH, 128), cntb)
    na, cntd = _pass_s(ef.reshape(2 * NH, 128), hyper2, node_idx,
                       ones_k, zeros_cb)
    cntd = cntd.reshape(NC, CNT_PAD)

    xw2 = _layer2(na.reshape(2, N, 128), cntd, b1.reshape(1, DH), W2)

    uf2 = _pass_e(xw2, node_idx, hyper_idx)
    ef2 = _scale_e(uf2.reshape(2, NH, 128), cntb)
    na2 = _pass_e(ef2, hyper_idx, node_idx)
    out = _final(na2.reshape(2, N, 128), cntd, b2.reshape(1, DOUT))
    return out
